# baseline (speedup 1.0000x reference)
import functools
import math

import jax
import jax.numpy as jnp
from jax import lax
from jax.experimental import pallas as pl
from jax.experimental.pallas import tpu as pltpu
from jax.experimental.pallas import tpu_sc as plsc

_SCALE = 0.125
_NUM_BUCKETS = 32
_MAX_DISTANCE = 128
_NSHIFT = 16


def _build_lut_body(tabT_ref, out_ref, *, seq, lut_len, cpad):
    nb2 = _NUM_BUCKETS // 2
    max_exact = nb2 // 2
    c = lax.broadcasted_iota(jnp.int32, (_NUM_BUCKETS, cpad), 1)
    b = lax.broadcasted_iota(jnp.int32, (_NUM_BUCKETS, cpad), 0)
    n = (seq - 1) - c
    base = jnp.where(n < 0, nb2, 0).astype(jnp.int32)
    a = jnp.abs(n)
    af = jnp.maximum(a, 1).astype(jnp.float32)
    vlarge = max_exact + (
        jnp.log(af / max_exact) / math.log(_MAX_DISTANCE / max_exact) * (nb2 - max_exact)
    ).astype(jnp.int32)
    vlarge = jnp.minimum(vlarge, nb2 - 1)
    bucket = base + jnp.where(a < max_exact, a, vlarge)
    onehot = (bucket == b).astype(jnp.float32)
    vals = lax.dot_general(
        tabT_ref[...], onehot, (((1,), (0,)), ((), ())),
        preferred_element_type=jnp.float32,
    ) * _SCALE
    for s in range(_NSHIFT):
        out_ref[s] = vals[:, s:s + lut_len]


def _lut_row_len(seq):
    return 2 * seq - _NSHIFT


def _build_lut(tableT, seq):
    heads = tableT.shape[0]
    lut_len = _lut_row_len(seq)
    cpad = ((lut_len + _NSHIFT + 127) // 128) * 128
    body = functools.partial(_build_lut_body, seq=seq, lut_len=lut_len, cpad=cpad)
    return pl.pallas_call(
        body,
        out_shape=jax.ShapeDtypeStruct((_NSHIFT, heads, lut_len), jnp.float32),
    )(tableT)


def _make_fanout(heads, seq):
    lut_len = _lut_row_len(seq)
    mesh = plsc.VectorSubcoreMesh(core_axis_name="c", subcore_axis_name="s")
    rows_per_tec = seq // 2
    rpb = 16
    nblk = rows_per_tec // rpb
    nbuf = 2

    @functools.partial(
        pl.kernel,
        mesh=mesh,
        out_type=jax.ShapeDtypeStruct((1, heads, 1, seq, seq), jnp.float32),
        scratch_types=[
            pltpu.VMEM((_NSHIFT * lut_len,), jnp.float32),
            pltpu.VMEM((nbuf, rpb, seq), jnp.float32),
            pltpu.SemaphoreType.DMA,
        ],
    )
    def fanout(lut_hbm, out_hbm, lut_tile, blk, sem):
        h = lax.axis_index("s")
        half = lax.axis_index("c")
        for s in range(_NSHIFT):
            pltpu.sync_copy(
                lut_hbm.at[pl.ds(pl.multiple_of((s * heads + h) * lut_len, _NSHIFT), lut_len)],
                lut_tile.at[pl.ds(s * lut_len, lut_len)],
            )
        i0 = half * rows_per_tec

        def blk_copy(m, b):
            i_start = pl.multiple_of(i0 + rpb * m, 8)
            return pltpu.make_async_copy(
                blk.at[b],
                out_hbm.at[0, h, 0, pl.ds(i_start, rpb), :],
                sem,
            )

        def build(m, b):
            cs0 = (seq - 1) - (i0 + rpb * m)
            t0 = cs0 - (rpb - 1)
            base = pl.multiple_of((rpb - 1) * lut_len + t0, _NSHIFT)
            dst = blk.at[b]

            @plsc.parallel_loop(0, seq // 16, unroll=4)
            def _col(u):
                c = u * 16
                for r in range(rpb):
                    vec = lut_tile[pl.ds(base + c - r * lut_len, 16)]
                    dst[r, pl.ds(c, 16)] = vec

        def body(m, carry):
            b = lax.rem(m, nbuf)

            @pl.when(m >= nbuf)
            def _():
                blk_copy(m - nbuf, b).wait()

            build(m, b)
            blk_copy(m, b).start()
            return carry

        lax.fori_loop(0, nblk, body, 0)

        def drain(m, carry):
            blk_copy(m, lax.rem(m, nbuf)).wait()
            return carry

        lax.fori_loop(nblk - nbuf, nblk, drain, 0)

    return fanout


def kernel(x, table):
    seq = x.shape[-2]
    heads = table.shape[1]
    tableT = jnp.transpose(table)
    lut = _build_lut(tableT, seq)
    return _make_fanout(heads, seq)(jnp.reshape(lut, (-1,)))


# speedup vs baseline: 1.0082x; 1.0082x over previous
import functools
import math

import jax
import jax.numpy as jnp
from jax import lax
from jax.experimental import pallas as pl
from jax.experimental.pallas import tpu as pltpu
from jax.experimental.pallas import tpu_sc as plsc

_SCALE = 0.125
_NUM_BUCKETS = 32
_MAX_DISTANCE = 128
_NSHIFT = 16


def _build_lut_body(tabT_ref, out_ref, *, seq, lut_len, cpad):
    nb2 = _NUM_BUCKETS // 2
    max_exact = nb2 // 2
    c = lax.broadcasted_iota(jnp.int32, (_NUM_BUCKETS, cpad), 1)
    b = lax.broadcasted_iota(jnp.int32, (_NUM_BUCKETS, cpad), 0)
    n = (seq - 1) - c
    base = jnp.where(n < 0, nb2, 0).astype(jnp.int32)
    a = jnp.abs(n)
    af = jnp.maximum(a, 1).astype(jnp.float32)
    vlarge = max_exact + (
        jnp.log(af / max_exact) / math.log(_MAX_DISTANCE / max_exact) * (nb2 - max_exact)
    ).astype(jnp.int32)
    vlarge = jnp.minimum(vlarge, nb2 - 1)
    bucket = base + jnp.where(a < max_exact, a, vlarge)
    onehot = (bucket == b).astype(jnp.float32)
    vals = lax.dot_general(
        tabT_ref[...], onehot, (((1,), (0,)), ((), ())),
        preferred_element_type=jnp.float32,
    ) * _SCALE
    for s in range(_NSHIFT):
        out_ref[s] = vals[:, s:s + lut_len]


def _build_lut(tableT, seq):
    heads = tableT.shape[0]
    lut_len = 2 * seq
    cpad = ((lut_len + _NSHIFT + 127) // 128) * 128
    body = functools.partial(_build_lut_body, seq=seq, lut_len=lut_len, cpad=cpad)
    return pl.pallas_call(
        body,
        out_shape=jax.ShapeDtypeStruct((_NSHIFT, heads, lut_len), jnp.float32),
    )(tableT)


def _make_fanout(heads, seq):
    lut_len = 2 * seq
    mesh = plsc.VectorSubcoreMesh(core_axis_name="c", subcore_axis_name="s")
    rows_per_tec = seq // 2
    nblk = rows_per_tec // 8
    nbuf = 2

    @functools.partial(
        pl.kernel,
        mesh=mesh,
        out_type=jax.ShapeDtypeStruct((1, heads, 1, seq, seq), jnp.float32),
        scratch_types=[
            pltpu.VMEM((_NSHIFT * lut_len,), jnp.float32),
            pltpu.VMEM((nbuf, 8, seq), jnp.float32),
            pltpu.SemaphoreType.DMA,
        ],
    )
    def fanout(lut_hbm, out_hbm, lut_tile, blk, sem):
        h = lax.axis_index("s")
        half = lax.axis_index("c")
        for s in range(_NSHIFT):
            pltpu.sync_copy(
                lut_hbm.at[pl.ds(pl.multiple_of((s * heads + h) * lut_len, lut_len), lut_len)],
                lut_tile.at[pl.ds(s * lut_len, lut_len)],
            )
        i0 = half * rows_per_tec

        def blk_copy(m, b):
            i_start = pl.multiple_of(i0 + 8 * m, 8)
            return pltpu.make_async_copy(
                blk.at[b],
                out_hbm.at[0, h, 0, pl.ds(i_start, 8), :],
                sem,
            )

        def build(m, b):
            cs0 = (seq - 1) - (i0 + 8 * m)
            e = lax.rem(cs0 - 7, _NSHIFT)
            t0 = cs0 - 7 - e
            base = pl.multiple_of((7 + e) * lut_len + t0, _NSHIFT)
            dst = blk.at[b]

            @plsc.parallel_loop(0, seq // 16, unroll=8)
            def _col(u):
                c = u * 16
                for r in range(8):
                    vec = lut_tile[pl.ds(base + c - r * lut_len, 16)]
                    dst[r, pl.ds(c, 16)] = vec

        def body(m, carry):
            b = lax.rem(m, nbuf)

            @pl.when(m >= nbuf)
            def _():
                blk_copy(m - nbuf, b).wait()

            build(m, b)
            blk_copy(m, b).start()
            return carry

        lax.fori_loop(0, nblk, body, 0)

        def drain(m, carry):
            blk_copy(m, lax.rem(m, nbuf)).wait()
            return carry

        lax.fori_loop(nblk - nbuf, nblk, drain, 0)

    return fanout


def kernel(x, table):
    seq = x.shape[-2]
    heads = table.shape[1]
    tableT = jnp.transpose(table)
    lut = _build_lut(tableT, seq)
    return _make_fanout(heads, seq)(jnp.reshape(lut, (-1,)))
